# baseline (speedup 1.0000x reference)
import functools

import jax
import jax.numpy as jnp
from jax import lax
from jax.experimental import pallas as pl
from jax.experimental.pallas import tpu as pltpu
from jax.experimental.pallas import tpu_sc as plsc

N = 50000
F = 256
G = 1024
R1 = 1000
NB1 = N // R1

NW = 32
C = 1568
NPAD = NW * C
NBW = NPAD // C
SEGW = G // NW
CH = 16
SLEN = 1040

_SELU_A = 1.6732632423543772
_SELU_S = 1.0507009873554805
_NEG = -1e30


def _selu(x):
    return _SELU_S * jnp.where(x > 0, x, _SELU_A * (jnp.exp(x) - 1.0))


def _mlp_body(x_ref, gw1_ref, gb1_ref, gw2_ref, gb2_ref,
              mw1_ref, mb1_ref, mw2_ref, mb2_ref, gate_ref, msg_ref):
    x = x_ref[...]
    h = _selu(jnp.dot(x, gw1_ref[...], preferred_element_type=jnp.float32)
              + gb1_ref[...])
    gate_ref[...] = (jnp.dot(h, gw2_ref[...], preferred_element_type=jnp.float32)
                     + gb2_ref[...])
    m = _selu(jnp.dot(x, mw1_ref[...], preferred_element_type=jnp.float32)
              + mb1_ref[...])
    msg_ref[...] = _selu(jnp.dot(m, mw2_ref[...],
                                 preferred_element_type=jnp.float32)
                         + mb2_ref[...])


def _segmax_body(batch_ref, gate_ref, gmax_ref, starts_ref, gmax_s, cnt_s):
    b = pl.program_id(0)

    @pl.when(b == 0)
    def _init():
        gmax_s[...] = jnp.full((1, G), _NEG, jnp.float32)
        cnt_s[...] = jnp.zeros((1, G), jnp.int32)

    ids = batch_ref[...]
    iot = lax.broadcasted_iota(jnp.int32, (R1, G), 1)
    oh = ids == iot
    vals = jnp.where(oh, gate_ref[...], _NEG)
    gmax_s[...] = jnp.maximum(gmax_s[...], jnp.max(vals, axis=0, keepdims=True))
    lt = (ids < iot).astype(jnp.int32)
    cnt_s[...] += jnp.sum(lt, axis=0, keepdims=True)

    @pl.when(b == NB1 - 1)
    def _flush():
        gmax_ref[...] = gmax_s[...]
        starts_ref[...] = cnt_s[...]


def _ew_body(batch_ref, gate_ref, msg_ref, gmax_ref, w_ref, e_ref):
    b = pl.program_id(0)
    ids = batch_ref[...]
    rows = lax.broadcasted_iota(jnp.int32, (C, 1), 0) + b * C
    valid = rows < N
    oh = jnp.logical_and(
        ids == lax.broadcasted_iota(jnp.int32, (C, G), 1), valid)
    gmaxg = jnp.max(jnp.where(oh, gmax_ref[...], _NEG), axis=1, keepdims=True)
    e = jnp.exp(jnp.minimum(gate_ref[...] - gmaxg, 0.0))
    e = jnp.where(valid, e, 0.0)
    w_ref[...] = jnp.where(valid, e * msg_ref[...], 0.0)
    e_ref[...] = jnp.broadcast_to(e, (C, 16))


def _sc_ctx_body(starts_hbm, w_hbm, e_hbm, ctx_out,
                 starts_v, w_v, e_v, stage_v):
    cid = lax.axis_index("c")
    sid = lax.axis_index("s")
    wid = sid * 2 + cid
    g0 = pl.multiple_of(wid * SEGW, 8)
    pltpu.sync_copy(starts_hbm.at[pl.ds(g0, 48)], starts_v)
    def seg_body(s, carry):
        sv = starts_v[pl.ds(s, 16)]
        start = sv[0]
        end = sv[1]
        start_al = (start // 8) * 8
        nch = (end - start_al + (CH - 1)) // CH
        zero = jnp.zeros((16,), jnp.float32)
        init = tuple([zero] * 16) + (zero,)

        def chunk_body(c, acc):
            accs = list(acc[:16])
            ea = acc[16]
            off = pl.multiple_of(start_al + c * CH, 8)
            pltpu.sync_copy(w_hbm.at[pl.ds(off, CH)], w_v)
            pltpu.sync_copy(e_hbm.at[pl.ds(off, CH)], e_v)
            for r in range(CH):
                vm = jnp.where(
                    jnp.logical_and(off + r >= start, off + r < end), 1.0, 0.0)
                for k in range(16):
                    accs[k] = accs[k] + w_v[r, pl.ds(16 * k, 16)] * vm
                ea = ea + e_v[r, :] * vm
            return tuple(accs) + (ea,)

        res = lax.fori_loop(0, nch, chunk_body, init)
        ea = jnp.maximum(res[16], 1e-30)
        for k in range(16):
            stage_v[s, pl.ds(16 * k, 16)] = res[k] / ea
        return carry

    lax.fori_loop(0, SEGW, seg_body, 0)
    pltpu.sync_copy(stage_v, ctx_out.at[pl.ds(g0, SEGW)])


def _gather_ln_body(batch_ref, x_ref, ctx_ref, ln_g_ref, ln_b_ref, out_ref):
    ids = batch_ref[...]
    ohf = (ids == lax.broadcasted_iota(jnp.int32, (R1, G), 1)
           ).astype(jnp.float32)
    gathered = jnp.dot(ohf, ctx_ref[...], preferred_element_type=jnp.float32)
    u = x_ref[...] + gathered
    mean = jnp.mean(u, axis=1, keepdims=True)
    d = u - mean
    var = jnp.mean(d * d, axis=1, keepdims=True)
    out_ref[...] = (d * lax.rsqrt(var + 1e-5)) * ln_g_ref[...] + ln_b_ref[...]


def _whole(shape):
    return pl.BlockSpec(shape, lambda b: tuple(0 for _ in shape))


def kernel(elem_weights, elem_in_fea, batch, gw1, gb1, gw2, gb2,
           mw1, mb1, mw2, mb2, ln_g, ln_b):
    del elem_weights
    x = elem_in_fea
    batch_col = batch.astype(jnp.int32).reshape(N, 1)

    gate, msg = pl.pallas_call(
        _mlp_body,
        grid=(NB1,),
        in_specs=[
            pl.BlockSpec((R1, F), lambda b: (b, 0)),
            _whole((F, 256)), _whole((1, 256)),
            _whole((256, 1)), _whole((1, 1)),
            _whole((F, 256)), _whole((1, 256)),
            _whole((256, F)), _whole((1, F)),
        ],
        out_specs=[
            pl.BlockSpec((R1, 1), lambda b: (b, 0)),
            pl.BlockSpec((R1, F), lambda b: (b, 0)),
        ],
        out_shape=[
            jax.ShapeDtypeStruct((N, 1), jnp.float32),
            jax.ShapeDtypeStruct((N, F), jnp.float32),
        ],
        compiler_params=pltpu.CompilerParams(
            dimension_semantics=("parallel",)),
    )(x, gw1, gb1.reshape(1, -1), gw2, gb2.reshape(1, -1),
      mw1, mb1.reshape(1, -1), mw2, mb2.reshape(1, -1))

    gmax, starts2d = pl.pallas_call(
        _segmax_body,
        grid=(NB1,),
        in_specs=[
            pl.BlockSpec((R1, 1), lambda b: (b, 0)),
            pl.BlockSpec((R1, 1), lambda b: (b, 0)),
        ],
        out_specs=[_whole((1, G)), _whole((1, G))],
        out_shape=[jax.ShapeDtypeStruct((1, G), jnp.float32),
                   jax.ShapeDtypeStruct((1, G), jnp.int32)],
        scratch_shapes=[pltpu.VMEM((1, G), jnp.float32),
                        pltpu.VMEM((1, G), jnp.int32)],
        compiler_params=pltpu.CompilerParams(
            dimension_semantics=("arbitrary",)),
    )(batch_col, gate)

    starts_ext = jnp.concatenate(
        [starts2d.reshape(G), jnp.full((SLEN - G,), N, jnp.int32)])

    w, e_rep = pl.pallas_call(
        _ew_body,
        grid=(NBW,),
        in_specs=[
            pl.BlockSpec((C, 1), lambda b: (b, 0)),
            pl.BlockSpec((C, 1), lambda b: (b, 0)),
            pl.BlockSpec((C, F), lambda b: (b, 0)),
            _whole((1, G)),
        ],
        out_specs=[
            pl.BlockSpec((C, F), lambda b: (b, 0)),
            pl.BlockSpec((C, 16), lambda b: (b, 0)),
        ],
        out_shape=[
            jax.ShapeDtypeStruct((NPAD, F), jnp.float32),
            jax.ShapeDtypeStruct((NPAD, 16), jnp.float32),
        ],
        compiler_params=pltpu.CompilerParams(
            dimension_semantics=("parallel",)),
    )(batch_col, gate, msg, gmax)

    ctx = pl.kernel(
        _sc_ctx_body,
        mesh=plsc.VectorSubcoreMesh(core_axis_name="c", subcore_axis_name="s"),
        out_type=jax.ShapeDtypeStruct((G, F), jnp.float32),
        scratch_types=[
            pltpu.VMEM((48,), jnp.int32),
            pltpu.VMEM((CH, F), jnp.float32),
            pltpu.VMEM((CH, 16), jnp.float32),
            pltpu.VMEM((SEGW, F), jnp.float32),
        ],
    )(starts_ext, w, e_rep)

    out = pl.pallas_call(
        _gather_ln_body,
        grid=(NB1,),
        in_specs=[
            pl.BlockSpec((R1, 1), lambda b: (b, 0)),
            pl.BlockSpec((R1, F), lambda b: (b, 0)),
            _whole((G, F)),
            _whole((1, F)),
            _whole((1, F)),
        ],
        out_specs=pl.BlockSpec((R1, F), lambda b: (b, 0)),
        out_shape=jax.ShapeDtypeStruct((N, F), jnp.float32),
        compiler_params=pltpu.CompilerParams(
            dimension_semantics=("parallel",)),
    )(batch_col, x, ctx, ln_g.reshape(1, -1), ln_b.reshape(1, -1))

    return out


# speedup vs baseline: 1.0162x; 1.0162x over previous
import functools

import jax
import jax.numpy as jnp
from jax import lax
from jax.experimental import pallas as pl
from jax.experimental.pallas import tpu as pltpu
from jax.experimental.pallas import tpu_sc as plsc

N = 50000
F = 256
G = 1024
R1 = 1000
NB1 = N // R1

NW = 32
C = 1568
NPAD = NW * C
NBW = NPAD // C
SEGW = G // NW
CH = 48
FE = F + 16
SLEN = 1040

_SELU_A = 1.6732632423543772
_SELU_S = 1.0507009873554805
_NEG = -1e30


def _selu(x):
    return _SELU_S * jnp.where(x > 0, x, _SELU_A * (jnp.exp(x) - 1.0))


def _mlp_body(x_ref, gw1_ref, gb1_ref, gw2_ref, gb2_ref,
              mw1_ref, mb1_ref, mw2_ref, mb2_ref, gate_ref, msg_ref):
    x = x_ref[...]
    h = _selu(jnp.dot(x, gw1_ref[...], preferred_element_type=jnp.float32)
              + gb1_ref[...])
    gate_ref[...] = (jnp.dot(h, gw2_ref[...], preferred_element_type=jnp.float32)
                     + gb2_ref[...])
    m = _selu(jnp.dot(x, mw1_ref[...], preferred_element_type=jnp.float32)
              + mb1_ref[...])
    msg_ref[...] = _selu(jnp.dot(m, mw2_ref[...],
                                 preferred_element_type=jnp.float32)
                         + mb2_ref[...])


def _segmax_body(batch_ref, gate_ref, gmax_ref, starts_ref, gmax_s, cnt_s):
    b = pl.program_id(0)

    @pl.when(b == 0)
    def _init():
        gmax_s[...] = jnp.full((1, G), _NEG, jnp.float32)
        cnt_s[...] = jnp.zeros((1, G), jnp.int32)

    ids = batch_ref[...]
    iot = lax.broadcasted_iota(jnp.int32, (R1, G), 1)
    oh = ids == iot
    vals = jnp.where(oh, gate_ref[...], _NEG)
    gmax_s[...] = jnp.maximum(gmax_s[...], jnp.max(vals, axis=0, keepdims=True))
    lt = (ids < iot).astype(jnp.int32)
    cnt_s[...] += jnp.sum(lt, axis=0, keepdims=True)

    @pl.when(b == NB1 - 1)
    def _flush():
        gmax_ref[...] = gmax_s[...]
        starts_ref[...] = cnt_s[...]


def _ew_body(batch_ref, gate_ref, msg_ref, gmax_ref, w_ref):
    b = pl.program_id(0)
    ids = batch_ref[...]
    rows = lax.broadcasted_iota(jnp.int32, (C, 1), 0) + b * C
    valid = rows < N
    oh = jnp.logical_and(
        ids == lax.broadcasted_iota(jnp.int32, (C, G), 1), valid)
    gmaxg = jnp.max(jnp.where(oh, gmax_ref[...], _NEG), axis=1, keepdims=True)
    e = jnp.exp(jnp.minimum(gate_ref[...] - gmaxg, 0.0))
    e = jnp.where(valid, e, 0.0)
    w_ref[:, :F] = jnp.where(valid, e * msg_ref[...], 0.0)
    w_ref[:, F:] = jnp.broadcast_to(e, (C, 16))


def _sc_ctx_body(starts_hbm, w_hbm, ctx_out, starts_v, w_v, stage_v):
    cid = lax.axis_index("c")
    sid = lax.axis_index("s")
    wid = sid * 2 + cid
    g0 = pl.multiple_of(wid * SEGW, 8)
    pltpu.sync_copy(starts_hbm.at[pl.ds(g0, 48)], starts_v)
    def seg_body(s, carry):
        sv = starts_v[pl.ds(s, 16)]
        start = sv[0]
        end = sv[1]
        start_al = (start // 8) * 8
        nch = (end - start_al + (CH - 1)) // CH
        zero = jnp.zeros((16,), jnp.float32)
        init = tuple([zero] * 17)

        def chunk_body(c, acc):
            accs = list(acc)
            off = pl.multiple_of(start_al + c * CH, 8)
            pltpu.sync_copy(w_hbm.at[pl.ds(off, CH)], w_v)
            for r in range(CH):
                vm = jnp.where(
                    jnp.logical_and(off + r >= start, off + r < end), 1.0, 0.0)
                for k in range(17):
                    accs[k] = accs[k] + w_v[r, pl.ds(16 * k, 16)] * vm
            return tuple(accs)

        res = lax.fori_loop(0, nch, chunk_body, init)
        ea = jnp.maximum(res[16], 1e-30)
        for k in range(16):
            stage_v[s, pl.ds(16 * k, 16)] = res[k] / ea
        return carry

    lax.fori_loop(0, SEGW, seg_body, 0)
    pltpu.sync_copy(stage_v, ctx_out.at[pl.ds(g0, SEGW)])


def _gather_ln_body(batch_ref, x_ref, ctx_ref, ln_g_ref, ln_b_ref, out_ref):
    ids = batch_ref[...]
    ohb = (ids == lax.broadcasted_iota(jnp.int32, (R1, G), 1)
           ).astype(jnp.bfloat16)
    gathered = jnp.dot(ohb, ctx_ref[...].astype(jnp.bfloat16),
                       preferred_element_type=jnp.float32)
    u = x_ref[...] + gathered
    mean = jnp.mean(u, axis=1, keepdims=True)
    d = u - mean
    var = jnp.mean(d * d, axis=1, keepdims=True)
    out_ref[...] = (d * lax.rsqrt(var + 1e-5)) * ln_g_ref[...] + ln_b_ref[...]


def _whole(shape):
    return pl.BlockSpec(shape, lambda b: tuple(0 for _ in shape))


def kernel(elem_weights, elem_in_fea, batch, gw1, gb1, gw2, gb2,
           mw1, mb1, mw2, mb2, ln_g, ln_b):
    del elem_weights
    x = elem_in_fea
    batch_col = batch.astype(jnp.int32).reshape(N, 1)

    gate, msg = pl.pallas_call(
        _mlp_body,
        grid=(NB1,),
        in_specs=[
            pl.BlockSpec((R1, F), lambda b: (b, 0)),
            _whole((F, 256)), _whole((1, 256)),
            _whole((256, 1)), _whole((1, 1)),
            _whole((F, 256)), _whole((1, 256)),
            _whole((256, F)), _whole((1, F)),
        ],
        out_specs=[
            pl.BlockSpec((R1, 1), lambda b: (b, 0)),
            pl.BlockSpec((R1, F), lambda b: (b, 0)),
        ],
        out_shape=[
            jax.ShapeDtypeStruct((N, 1), jnp.float32),
            jax.ShapeDtypeStruct((N, F), jnp.float32),
        ],
        compiler_params=pltpu.CompilerParams(
            dimension_semantics=("parallel",)),
    )(x, gw1, gb1.reshape(1, -1), gw2, gb2.reshape(1, -1),
      mw1, mb1.reshape(1, -1), mw2, mb2.reshape(1, -1))

    gmax, starts2d = pl.pallas_call(
        _segmax_body,
        grid=(NB1,),
        in_specs=[
            pl.BlockSpec((R1, 1), lambda b: (b, 0)),
            pl.BlockSpec((R1, 1), lambda b: (b, 0)),
        ],
        out_specs=[_whole((1, G)), _whole((1, G))],
        out_shape=[jax.ShapeDtypeStruct((1, G), jnp.float32),
                   jax.ShapeDtypeStruct((1, G), jnp.int32)],
        scratch_shapes=[pltpu.VMEM((1, G), jnp.float32),
                        pltpu.VMEM((1, G), jnp.int32)],
        compiler_params=pltpu.CompilerParams(
            dimension_semantics=("arbitrary",)),
    )(batch_col, gate)

    starts_ext = jnp.concatenate(
        [starts2d.reshape(G), jnp.full((SLEN - G,), N, jnp.int32)])

    w = pl.pallas_call(
        _ew_body,
        grid=(NBW,),
        in_specs=[
            pl.BlockSpec((C, 1), lambda b: (b, 0)),
            pl.BlockSpec((C, 1), lambda b: (b, 0)),
            pl.BlockSpec((C, F), lambda b: (b, 0)),
            _whole((1, G)),
        ],
        out_specs=pl.BlockSpec((C, FE), lambda b: (b, 0)),
        out_shape=jax.ShapeDtypeStruct((NPAD, FE), jnp.float32),
        compiler_params=pltpu.CompilerParams(
            dimension_semantics=("parallel",)),
    )(batch_col, gate, msg, gmax)

    ctx = pl.kernel(
        _sc_ctx_body,
        mesh=plsc.VectorSubcoreMesh(core_axis_name="c", subcore_axis_name="s"),
        out_type=jax.ShapeDtypeStruct((G, F), jnp.float32),
        scratch_types=[
            pltpu.VMEM((48,), jnp.int32),
            pltpu.VMEM((CH, FE), jnp.float32),
            pltpu.VMEM((SEGW, F), jnp.float32),
        ],
    )(starts_ext, w)

    out = pl.pallas_call(
        _gather_ln_body,
        grid=(NB1,),
        in_specs=[
            pl.BlockSpec((R1, 1), lambda b: (b, 0)),
            pl.BlockSpec((R1, F), lambda b: (b, 0)),
            _whole((G, F)),
            _whole((1, F)),
            _whole((1, F)),
        ],
        out_specs=pl.BlockSpec((R1, F), lambda b: (b, 0)),
        out_shape=jax.ShapeDtypeStruct((N, F), jnp.float32),
        compiler_params=pltpu.CompilerParams(
            dimension_semantics=("parallel",)),
    )(batch_col, x, ctx, ln_g.reshape(1, -1), ln_b.reshape(1, -1))

    return out


# speedup vs baseline: 1.6455x; 1.6193x over previous
import functools

import jax
import jax.numpy as jnp
from jax.experimental import pallas as pl
from jax.experimental.pallas import tpu as pltpu

N = 50000
F = 256
G = 1024
R1 = 1000
NB1 = N // R1
RB = 1000
NB = N // RB

_SELU_A = 1.6732632423543772
_SELU_S = 1.0507009873554805
_NEG = -1e30


def _selu(x):
    return _SELU_S * jnp.where(x > 0, x, _SELU_A * (jnp.exp(x) - 1.0))


def _mlp_body(x_ref, gw1_ref, gb1_ref, gw2_ref, gb2_ref,
              mw1_ref, mb1_ref, mw2_ref, mb2_ref, gate_ref, msg_ref):
    x = x_ref[...]
    h = _selu(jnp.dot(x, gw1_ref[...], preferred_element_type=jnp.float32)
              + gb1_ref[...])
    g = (jnp.dot(h, gw2_ref[...], preferred_element_type=jnp.float32)
         + gb2_ref[...])
    gate_ref[...] = g.reshape(1, R1, 1)
    m = _selu(jnp.dot(x, mw1_ref[...], preferred_element_type=jnp.float32)
              + mb1_ref[...])
    msg_ref[...] = _selu(jnp.dot(m, mw2_ref[...],
                                 preferred_element_type=jnp.float32)
                         + mb2_ref[...])


def _segmax_body(batch_ref, gate_ref, gmax_ref, gmax_s):
    b = pl.program_id(0)

    @pl.when(b == 0)
    def _init():
        gmax_s[...] = jnp.full((1, G), _NEG, jnp.float32)

    ids = batch_ref[0]
    oh = ids == jax.lax.broadcasted_iota(jnp.int32, (RB, G), 1)
    vals = jnp.where(oh, gate_ref[0], _NEG)
    gmax_s[...] = jnp.maximum(gmax_s[...], jnp.max(vals, axis=0, keepdims=True))

    @pl.when(b == NB - 1)
    def _flush():
        gmax_ref[...] = gmax_s[...]


def _scatter_body(batch_ref, gate_ref, msg_ref, gmax_ref, ctx_ref, s1_s, s2_s):
    b = pl.program_id(0)

    @pl.when(b == 0)
    def _init():
        s1_s[...] = jnp.zeros((G, 1), jnp.float32)
        s2_s[...] = jnp.zeros((G, F), jnp.float32)

    ids = batch_ref[0]
    oh = ids == jax.lax.broadcasted_iota(jnp.int32, (RB, G), 1)
    ohb = oh.astype(jnp.bfloat16)
    gmaxg = jnp.max(jnp.where(oh, gmax_ref[...], _NEG), axis=1, keepdims=True)
    e = jnp.exp(jnp.minimum(gate_ref[0] - gmaxg, 0.0))
    w = e * msg_ref[...]
    dn = (((0,), (0,)), ((), ()))
    s1_s[...] += jax.lax.dot_general(ohb, e.astype(jnp.bfloat16), dn,
                                     preferred_element_type=jnp.float32)
    s2_s[...] += jax.lax.dot_general(ohb, w.astype(jnp.bfloat16), dn,
                                     preferred_element_type=jnp.float32)

    @pl.when(b == NB - 1)
    def _flush():
        ctx_ref[...] = s2_s[...] / jnp.maximum(s1_s[...], 1e-30)


def _gather_ln_body(batch_ref, x_ref, ctx_ref, ln_g_ref, ln_b_ref, out_ref):
    ids = batch_ref[0]
    ohb = (ids == jax.lax.broadcasted_iota(jnp.int32, (RB, G), 1)
           ).astype(jnp.bfloat16)
    gathered = jnp.dot(ohb, ctx_ref[...].astype(jnp.bfloat16),
                       preferred_element_type=jnp.float32)
    u = x_ref[...] + gathered
    mean = jnp.mean(u, axis=1, keepdims=True)
    d = u - mean
    var = jnp.mean(d * d, axis=1, keepdims=True)
    out_ref[...] = (d * jax.lax.rsqrt(var + 1e-5)) * ln_g_ref[...] + ln_b_ref[...]


def _whole(shape):
    return pl.BlockSpec(shape, lambda b: tuple(0 for _ in shape))


def kernel(elem_weights, elem_in_fea, batch, gw1, gb1, gw2, gb2,
           mw1, mb1, mw2, mb2, ln_g, ln_b):
    del elem_weights
    x = elem_in_fea
    batch3 = batch.astype(jnp.int32).reshape(NB, RB, 1)

    gate, msg = pl.pallas_call(
        _mlp_body,
        grid=(NB1,),
        in_specs=[
            pl.BlockSpec((R1, F), lambda b: (b, 0)),
            _whole((F, 256)), _whole((1, 256)),
            _whole((256, 1)), _whole((1, 1)),
            _whole((F, 256)), _whole((1, 256)),
            _whole((256, F)), _whole((1, F)),
        ],
        out_specs=[
            pl.BlockSpec((1, R1, 1), lambda b: (b, 0, 0)),
            pl.BlockSpec((R1, F), lambda b: (b, 0)),
        ],
        out_shape=[
            jax.ShapeDtypeStruct((NB1, R1, 1), jnp.float32),
            jax.ShapeDtypeStruct((N, F), jnp.float32),
        ],
        compiler_params=pltpu.CompilerParams(
            dimension_semantics=("parallel",)),
    )(x, gw1, gb1.reshape(1, -1), gw2, gb2.reshape(1, -1),
      mw1, mb1.reshape(1, -1), mw2, mb2.reshape(1, -1))
    gate3 = gate

    gmax = pl.pallas_call(
        _segmax_body,
        grid=(NB,),
        in_specs=[
            pl.BlockSpec((1, RB, 1), lambda b: (b, 0, 0)),
            pl.BlockSpec((1, RB, 1), lambda b: (b, 0, 0)),
        ],
        out_specs=_whole((1, G)),
        out_shape=jax.ShapeDtypeStruct((1, G), jnp.float32),
        scratch_shapes=[pltpu.VMEM((1, G), jnp.float32)],
        compiler_params=pltpu.CompilerParams(
            dimension_semantics=("arbitrary",)),
    )(batch3, gate3)

    ctx = pl.pallas_call(
        _scatter_body,
        grid=(NB,),
        in_specs=[
            pl.BlockSpec((1, RB, 1), lambda b: (b, 0, 0)),
            pl.BlockSpec((1, RB, 1), lambda b: (b, 0, 0)),
            pl.BlockSpec((RB, F), lambda b: (b, 0)),
            _whole((1, G)),
        ],
        out_specs=_whole((G, F)),
        out_shape=jax.ShapeDtypeStruct((G, F), jnp.float32),
        scratch_shapes=[pltpu.VMEM((G, 1), jnp.float32),
                        pltpu.VMEM((G, F), jnp.float32)],
        compiler_params=pltpu.CompilerParams(
            dimension_semantics=("arbitrary",)),
    )(batch3, gate3, msg, gmax)

    out = pl.pallas_call(
        _gather_ln_body,
        grid=(NB,),
        in_specs=[
            pl.BlockSpec((1, RB, 1), lambda b: (b, 0, 0)),
            pl.BlockSpec((RB, F), lambda b: (b, 0)),
            _whole((G, F)),
            _whole((1, F)),
            _whole((1, F)),
        ],
        out_specs=pl.BlockSpec((RB, F), lambda b: (b, 0)),
        out_shape=jax.ShapeDtypeStruct((N, F), jnp.float32),
        compiler_params=pltpu.CompilerParams(
            dimension_semantics=("parallel",)),
    )(batch3, x, ctx, ln_g.reshape(1, -1), ln_b.reshape(1, -1))

    return out


# speedup vs baseline: 1.8961x; 1.1523x over previous
import functools

import jax
import jax.numpy as jnp
from jax.experimental import pallas as pl
from jax.experimental.pallas import tpu as pltpu

N = 50000
F = 256
G = 1024
R1 = 1000
NB1 = N // R1
RB = 1000
NB = N // RB

_SELU_A = 1.6732632423543772
_SELU_S = 1.0507009873554805
_NEG = -1e30


def _selu(x):
    return _SELU_S * jnp.where(x > 0, x, _SELU_A * (jnp.exp(x) - 1.0))


def _gate_body(batch_ref, x_ref, gw1_ref, gb1_ref, gw2_ref, gb2_ref,
               gate_ref, gmax_ref, gmax_s):
    b = pl.program_id(0)

    @pl.when(b == 0)
    def _init():
        gmax_s[...] = jnp.full((1, G), _NEG, jnp.float32)

    x = x_ref[...]
    h = _selu(jnp.dot(x, gw1_ref[...], preferred_element_type=jnp.float32)
              + gb1_ref[...])
    g = (jnp.dot(h, gw2_ref[...], preferred_element_type=jnp.float32)
         + gb2_ref[...])
    gate_ref[...] = g.reshape(1, R1, 1)
    ids = batch_ref[0]
    oh = ids == jax.lax.broadcasted_iota(jnp.int32, (R1, G), 1)
    vals = jnp.where(oh, g, _NEG)
    gmax_s[...] = jnp.maximum(gmax_s[...], jnp.max(vals, axis=0, keepdims=True))

    @pl.when(b == NB - 1)
    def _flush():
        gmax_ref[...] = gmax_s[...]


def _scatter_body(batch_ref, gate_ref, x_ref, mw1_ref, mb1_ref, mw2_ref,
                  mb2_ref, gmax_ref, ctx_ref, s1_s, s2_s):
    b = pl.program_id(0)

    @pl.when(b == 0)
    def _init():
        s1_s[...] = jnp.zeros((G, 1), jnp.float32)
        s2_s[...] = jnp.zeros((G, F), jnp.float32)

    ids = batch_ref[0]
    oh = ids == jax.lax.broadcasted_iota(jnp.int32, (RB, G), 1)
    ohb = oh.astype(jnp.bfloat16)
    gmaxg = jnp.max(jnp.where(oh, gmax_ref[...], _NEG), axis=1, keepdims=True)
    e = jnp.exp(jnp.minimum(gate_ref[0] - gmaxg, 0.0))
    m = _selu(jnp.dot(x_ref[...], mw1_ref[...],
                      preferred_element_type=jnp.float32) + mb1_ref[...])
    msg = _selu(jnp.dot(m, mw2_ref[...],
                        preferred_element_type=jnp.float32) + mb2_ref[...])
    w = e * msg
    dn = (((0,), (0,)), ((), ()))
    s1_s[...] += jax.lax.dot_general(ohb, e.astype(jnp.bfloat16), dn,
                                     preferred_element_type=jnp.float32)
    s2_s[...] += jax.lax.dot_general(ohb, w.astype(jnp.bfloat16), dn,
                                     preferred_element_type=jnp.float32)

    @pl.when(b == NB - 1)
    def _flush():
        ctx_ref[...] = s2_s[...] / jnp.maximum(s1_s[...], 1e-30)


def _gather_ln_body(batch_ref, x_ref, ctx_ref, ln_g_ref, ln_b_ref, out_ref):
    ids = batch_ref[0]
    ohb = (ids == jax.lax.broadcasted_iota(jnp.int32, (RB, G), 1)
           ).astype(jnp.bfloat16)
    gathered = jnp.dot(ohb, ctx_ref[...].astype(jnp.bfloat16),
                       preferred_element_type=jnp.float32)
    u = x_ref[...] + gathered
    mean = jnp.mean(u, axis=1, keepdims=True)
    d = u - mean
    var = jnp.mean(d * d, axis=1, keepdims=True)
    out_ref[...] = (d * jax.lax.rsqrt(var + 1e-5)) * ln_g_ref[...] + ln_b_ref[...]


def _whole(shape):
    return pl.BlockSpec(shape, lambda b: tuple(0 for _ in shape))


def kernel(elem_weights, elem_in_fea, batch, gw1, gb1, gw2, gb2,
           mw1, mb1, mw2, mb2, ln_g, ln_b):
    del elem_weights
    x = elem_in_fea
    batch3 = batch.astype(jnp.int32).reshape(NB, RB, 1)

    gate3, gmax = pl.pallas_call(
        _gate_body,
        grid=(NB,),
        in_specs=[
            pl.BlockSpec((1, RB, 1), lambda b: (b, 0, 0)),
            pl.BlockSpec((R1, F), lambda b: (b, 0)),
            _whole((F, 256)), _whole((1, 256)),
            _whole((256, 1)), _whole((1, 1)),
        ],
        out_specs=[
            pl.BlockSpec((1, R1, 1), lambda b: (b, 0, 0)),
            _whole((1, G)),
        ],
        out_shape=[
            jax.ShapeDtypeStruct((NB1, R1, 1), jnp.float32),
            jax.ShapeDtypeStruct((1, G), jnp.float32),
        ],
        scratch_shapes=[pltpu.VMEM((1, G), jnp.float32)],
        compiler_params=pltpu.CompilerParams(
            dimension_semantics=("arbitrary",)),
    )(batch3, x, gw1, gb1.reshape(1, -1), gw2, gb2.reshape(1, -1))

    ctx = pl.pallas_call(
        _scatter_body,
        grid=(NB,),
        in_specs=[
            pl.BlockSpec((1, RB, 1), lambda b: (b, 0, 0)),
            pl.BlockSpec((1, RB, 1), lambda b: (b, 0, 0)),
            pl.BlockSpec((RB, F), lambda b: (b, 0)),
            _whole((F, 256)), _whole((1, 256)),
            _whole((256, F)), _whole((1, F)),
            _whole((1, G)),
        ],
        out_specs=_whole((G, F)),
        out_shape=jax.ShapeDtypeStruct((G, F), jnp.float32),
        scratch_shapes=[pltpu.VMEM((G, 1), jnp.float32),
                        pltpu.VMEM((G, F), jnp.float32)],
        compiler_params=pltpu.CompilerParams(
            dimension_semantics=("arbitrary",)),
    )(batch3, gate3, x, mw1, mb1.reshape(1, -1), mw2, mb2.reshape(1, -1),
      gmax)

    out = pl.pallas_call(
        _gather_ln_body,
        grid=(NB,),
        in_specs=[
            pl.BlockSpec((1, RB, 1), lambda b: (b, 0, 0)),
            pl.BlockSpec((RB, F), lambda b: (b, 0)),
            _whole((G, F)),
            _whole((1, F)),
            _whole((1, F)),
        ],
        out_specs=pl.BlockSpec((RB, F), lambda b: (b, 0)),
        out_shape=jax.ShapeDtypeStruct((N, F), jnp.float32),
        compiler_params=pltpu.CompilerParams(
            dimension_semantics=("parallel",)),
    )(batch3, x, ctx, ln_g.reshape(1, -1), ln_b.reshape(1, -1))

    return out
